# baseline (speedup 1.0000x reference)
import functools

import jax
import jax.numpy as jnp
from jax import lax
from jax.experimental import pallas as pl
from jax.experimental.pallas import tpu as pltpu
from jax.experimental.pallas import tpu_sc as plsc

NUM_FACTORS = 32
BATCH = 16384
NUM_CORES = 2
NUM_SUBCORES = 16
NUM_WORKERS = NUM_CORES * NUM_SUBCORES
BPW = BATCH // NUM_WORKERS
LANES = 16
HALF = BPW // 2
HGROUPS = HALF // LANES

_mesh = plsc.VectorSubcoreMesh(core_axis_name="c", subcore_axis_name="s")


def _make_repack(n_rows: int, bk: int):
    grid = -(-n_rows // (4 * bk))

    def body(x, out):
        eye = jnp.eye(NUM_FACTORS, dtype=jnp.float32)
        dims = (((0,), (0,)), ((), ()))
        xt = lax.dot_general(x[...], eye, dims,
                             preferred_element_type=jnp.float32,
                             precision=lax.Precision.HIGHEST)
        out[...] = jnp.concatenate(
            [xt[s * bk:(s + 1) * bk, :] for s in range(4)], axis=1)

    return pl.pallas_call(
        body,
        grid=(grid,),
        in_specs=[pl.BlockSpec((NUM_FACTORS, 4 * bk), lambda g: (0, g))],
        out_specs=pl.BlockSpec((bk, 128), lambda g: (g, 0)),
        out_shape=jax.ShapeDtypeStruct((grid * bk, 128), jnp.float32),
    )


@functools.partial(
    pl.kernel,
    out_type=jax.ShapeDtypeStruct((BATCH,), jnp.float32),
    mesh=_mesh,
    scratch_types=[
        pltpu.VMEM((BPW,), jnp.int32),
        pltpu.VMEM((BPW,), jnp.int32),
        pltpu.VMEM((BPW,), jnp.int32),
        pltpu.VMEM((BPW,), jnp.int32),
        pltpu.VMEM((BPW,), jnp.int32),
        pltpu.VMEM((BPW,), jnp.int32),
        pltpu.VMEM((BPW,), jnp.int32),
        pltpu.VMEM((BPW,), jnp.int32),
        pltpu.VMEM((BPW,), jnp.int32),
        pltpu.VMEM((HALF, 128), jnp.float32),
        pltpu.VMEM((HALF, 128), jnp.float32),
        pltpu.VMEM((HALF, 128), jnp.float32),
        pltpu.VMEM((BPW,), jnp.float32),
        pltpu.VMEM((BPW,), jnp.float32),
        pltpu.VMEM((BPW,), jnp.float32),
        pltpu.VMEM((LANES,), jnp.float32),
        pltpu.VMEM((BPW,), jnp.float32),
        pltpu.VMEM((LANES * LANES,), jnp.float32),
        pltpu.SemaphoreType.DMA,
        pltpu.SemaphoreType.DMA,
    ],
    compiler_params=pltpu.CompilerParams(
        needs_layout_passes=False, use_tc_tiling_on_sc=False
    ),
)
def _mf_sc(r_inv, r_tic, r_dat, inv_b, tic_b, dat_b, gb, ki, kt, kd,
           si, st, sd, oi_h, ot_h, od_h, out, idx_inv, idx_tic, idx_dat,
           sub_inv, sub_tic, sub_dat, oidx_inv, oidx_tic, oidx_dat,
           rows_inv, rows_tic, rows_dat, bias_inv, bias_tic,
           bias_dat, gb_v, out_v, part_v, semb, semr):
    wid = lax.axis_index("s") * NUM_CORES + lax.axis_index("c")
    base = wid * BPW

    pltpu.sync_copy(ki.at[pl.ds(base, BPW)], idx_inv)
    pltpu.sync_copy(kt.at[pl.ds(base, BPW)], idx_tic)
    pltpu.sync_copy(kd.at[pl.ds(base, BPW)], idx_dat)
    pltpu.sync_copy(si.at[pl.ds(base, BPW)], sub_inv)
    pltpu.sync_copy(st.at[pl.ds(base, BPW)], sub_tic)
    pltpu.sync_copy(sd.at[pl.ds(base, BPW)], sub_dat)
    pltpu.sync_copy(oi_h.at[pl.ds(base, BPW)], oidx_inv)
    pltpu.sync_copy(ot_h.at[pl.ds(base, BPW)], oidx_tic)
    pltpu.sync_copy(od_h.at[pl.ds(base, BPW)], oidx_dat)
    pltpu.sync_copy(gb, gb_v)

    b1 = pltpu.async_copy(inv_b.at[oidx_inv], bias_inv, semb)
    b2 = pltpu.async_copy(tic_b.at[oidx_tic], bias_tic, semb)
    b3 = pltpu.async_copy(dat_b.at[oidx_dat], bias_dat, semb)

    gbv = gb_v[...]

    def do_half(h, carry):
        hbase = h * HALF
        g1 = pltpu.async_copy(
            r_inv.at[idx_inv.at[pl.ds(hbase, HALF)]], rows_inv, semr)
        g2 = pltpu.async_copy(
            r_tic.at[idx_tic.at[pl.ds(hbase, HALF)]], rows_tic, semr)
        g3 = pltpu.async_copy(
            r_dat.at[idx_dat.at[pl.ds(hbase, HALF)]], rows_dat, semr)
        g1.wait()
        g2.wait()
        g3.wait()

        lanes = lax.iota(jnp.int32, LANES)

        def group(g, c):
            o = g * LANES
            go = hbase + o
            sv_i = sub_inv[pl.ds(go, LANES)]
            sv_t = sub_tic[pl.ds(go, LANES)]
            sv_d = sub_dat[pl.ds(go, LANES)]
            for j in range(LANES):
                e = o + j
                jv = jnp.full((LANES,), j, jnp.int32)
                spl_i = jnp.take_along_axis(sv_i, jv, axis=0,
                                            mode="promise_in_bounds")
                spl_t = jnp.take_along_axis(sv_t, jv, axis=0,
                                            mode="promise_in_bounds")
                spl_d = jnp.take_along_axis(sv_d, jv, axis=0,
                                            mode="promise_in_bounds")

                def pick(rows, spl, off):
                    v = rows[e, pl.ds(off, LANES)]
                    for s in range(1, 4):
                        v = jnp.where(spl == s,
                                      rows[e, pl.ds(32 * s + off, LANES)], v)
                    return v

                a_lo = pick(rows_inv, spl_i, 0)
                a_hi = pick(rows_inv, spl_i, LANES)
                t_lo = pick(rows_tic, spl_t, 0)
                t_hi = pick(rows_tic, spl_t, LANES)
                d_lo = pick(rows_dat, spl_d, 0)
                d_hi = pick(rows_dat, spl_d, LANES)
                part_v[pl.ds(j * LANES, LANES)] = (
                    a_lo * t_lo * d_lo + a_hi * t_hi * d_hi)
            acc = bias_inv[pl.ds(go, LANES)] + bias_tic[pl.ds(go, LANES)]
            acc = acc + bias_dat[pl.ds(go, LANES)] + gbv
            rbase = lanes * LANES
            for col in range(LANES):
                acc = acc + plsc.load_gather(part_v, [rbase + col])
            out_v[pl.ds(go, LANES)] = acc
            return c

        lax.fori_loop(0, HGROUPS, group, 0)
        return carry

    b1.wait()
    b2.wait()
    b3.wait()
    lax.fori_loop(0, 2, do_half, 0)
    pltpu.sync_copy(out_v, out.at[pl.ds(base, BPW)])


def kernel(investor, ticker, date, investor_factors, ticker_factors,
           date_factors, investor_bias, ticker_bias, date_bias, global_bias):
    gb16 = jnp.broadcast_to(global_bias.astype(jnp.float32), (LANES,))
    inv = investor.astype(jnp.int32)
    tic = ticker.astype(jnp.int32)
    dat = date.astype(jnp.int32)

    def submap(i, bk_log2):
        s = (i >> bk_log2) & 3
        row = ((i >> (bk_log2 + 2)) << bk_log2) | (i & ((1 << bk_log2) - 1))
        return s, row

    s_i, ki = submap(inv, 11)
    s_t, kt = submap(tic, 11)
    s_d, kd = submap(dat, 8)

    r_inv = _make_repack(1000000, 2048)(investor_factors.T)
    r_tic = _make_repack(100000, 2048)(ticker_factors.T)
    r_dat = _make_repack(1000, 256)(date_factors.T)

    return _mf_sc(
        r_inv, r_tic, r_dat,
        investor_bias.reshape(-1),
        ticker_bias.reshape(-1),
        date_bias.reshape(-1),
        gb16,
        ki, kt, kd, s_i, s_t, s_d, inv, tic, dat,
    )


# speedup vs baseline: 1.5746x; 1.5746x over previous
import functools

import jax
import jax.numpy as jnp
from jax import lax
from jax.experimental import pallas as pl
from jax.experimental.pallas import tpu as pltpu
from jax.experimental.pallas import tpu_sc as plsc

NUM_FACTORS = 32
BATCH = 16384
NUM_CORES = 2
NUM_SUBCORES = 16
NUM_WORKERS = NUM_CORES * NUM_SUBCORES
BPW = BATCH // NUM_WORKERS
LANES = 16
HALF = BPW // 2
HGROUPS = HALF // LANES

_mesh = plsc.VectorSubcoreMesh(core_axis_name="c", subcore_axis_name="s")


def _make_repack(n_rows: int, bk: int):
    grid = -(-n_rows // (4 * bk))

    def body(x, out):
        xt = jnp.swapaxes(x[...], 0, 1)
        out[...] = jnp.concatenate(
            [xt[s * bk:(s + 1) * bk, :] for s in range(4)], axis=1)

    return pl.pallas_call(
        body,
        grid=(grid,),
        in_specs=[pl.BlockSpec((NUM_FACTORS, 4 * bk), lambda g: (0, g))],
        out_specs=pl.BlockSpec((bk, 128), lambda g: (g, 0)),
        out_shape=jax.ShapeDtypeStruct((grid * bk, 128), jnp.float32),
    )


@functools.partial(
    pl.kernel,
    out_type=jax.ShapeDtypeStruct((BATCH,), jnp.float32),
    mesh=_mesh,
    scratch_types=[
        pltpu.VMEM((BPW,), jnp.int32),
        pltpu.VMEM((BPW,), jnp.int32),
        pltpu.VMEM((BPW,), jnp.int32),
        pltpu.VMEM((BPW,), jnp.int32),
        pltpu.VMEM((BPW,), jnp.int32),
        pltpu.VMEM((BPW,), jnp.int32),
        pltpu.VMEM((BPW,), jnp.int32),
        pltpu.VMEM((BPW,), jnp.int32),
        pltpu.VMEM((BPW,), jnp.int32),
        pltpu.VMEM((HALF, 128), jnp.float32),
        pltpu.VMEM((HALF, 128), jnp.float32),
        pltpu.VMEM((HALF, 128), jnp.float32),
        pltpu.VMEM((BPW,), jnp.float32),
        pltpu.VMEM((BPW,), jnp.float32),
        pltpu.VMEM((BPW,), jnp.float32),
        pltpu.VMEM((LANES,), jnp.float32),
        pltpu.VMEM((BPW,), jnp.float32),
        pltpu.VMEM((LANES * LANES,), jnp.float32),
        pltpu.SemaphoreType.DMA,
        pltpu.SemaphoreType.DMA,
    ],
    compiler_params=pltpu.CompilerParams(
        needs_layout_passes=False, use_tc_tiling_on_sc=False
    ),
)
def _mf_sc(r_inv, r_tic, r_dat, inv_b, tic_b, dat_b, gb, ki, kt, kd,
           si, st, sd, oi_h, ot_h, od_h, out, idx_inv, idx_tic, idx_dat,
           sub_inv, sub_tic, sub_dat, oidx_inv, oidx_tic, oidx_dat,
           rows_inv, rows_tic, rows_dat, bias_inv, bias_tic,
           bias_dat, gb_v, out_v, part_v, semb, semr):
    wid = lax.axis_index("s") * NUM_CORES + lax.axis_index("c")
    base = wid * BPW

    pltpu.sync_copy(ki.at[pl.ds(base, BPW)], idx_inv)
    pltpu.sync_copy(kt.at[pl.ds(base, BPW)], idx_tic)
    pltpu.sync_copy(kd.at[pl.ds(base, BPW)], idx_dat)
    pltpu.sync_copy(si.at[pl.ds(base, BPW)], sub_inv)
    pltpu.sync_copy(st.at[pl.ds(base, BPW)], sub_tic)
    pltpu.sync_copy(sd.at[pl.ds(base, BPW)], sub_dat)
    pltpu.sync_copy(oi_h.at[pl.ds(base, BPW)], oidx_inv)
    pltpu.sync_copy(ot_h.at[pl.ds(base, BPW)], oidx_tic)
    pltpu.sync_copy(od_h.at[pl.ds(base, BPW)], oidx_dat)
    pltpu.sync_copy(gb, gb_v)

    b1 = pltpu.async_copy(inv_b.at[oidx_inv], bias_inv, semb)
    b2 = pltpu.async_copy(tic_b.at[oidx_tic], bias_tic, semb)
    b3 = pltpu.async_copy(dat_b.at[oidx_dat], bias_dat, semb)

    gbv = gb_v[...]

    def do_half(h, carry):
        hbase = h * HALF
        g1 = pltpu.async_copy(
            r_inv.at[idx_inv.at[pl.ds(hbase, HALF)]], rows_inv, semr)
        g2 = pltpu.async_copy(
            r_tic.at[idx_tic.at[pl.ds(hbase, HALF)]], rows_tic, semr)
        g3 = pltpu.async_copy(
            r_dat.at[idx_dat.at[pl.ds(hbase, HALF)]], rows_dat, semr)
        g1.wait()
        g2.wait()
        g3.wait()

        lanes = lax.iota(jnp.int32, LANES)

        def group(g, c):
            o = g * LANES
            go = hbase + o
            sv_i = sub_inv[pl.ds(go, LANES)]
            sv_t = sub_tic[pl.ds(go, LANES)]
            sv_d = sub_dat[pl.ds(go, LANES)]
            for j in range(LANES):
                e = o + j
                jv = jnp.full((LANES,), j, jnp.int32)
                spl_i = jnp.take_along_axis(sv_i, jv, axis=0,
                                            mode="promise_in_bounds")
                spl_t = jnp.take_along_axis(sv_t, jv, axis=0,
                                            mode="promise_in_bounds")
                spl_d = jnp.take_along_axis(sv_d, jv, axis=0,
                                            mode="promise_in_bounds")

                def pick(rows, spl, off):
                    v = rows[e, pl.ds(off, LANES)]
                    for s in range(1, 4):
                        v = jnp.where(spl == s,
                                      rows[e, pl.ds(32 * s + off, LANES)], v)
                    return v

                a_lo = pick(rows_inv, spl_i, 0)
                a_hi = pick(rows_inv, spl_i, LANES)
                t_lo = pick(rows_tic, spl_t, 0)
                t_hi = pick(rows_tic, spl_t, LANES)
                d_lo = pick(rows_dat, spl_d, 0)
                d_hi = pick(rows_dat, spl_d, LANES)
                part_v[pl.ds(j * LANES, LANES)] = (
                    a_lo * t_lo * d_lo + a_hi * t_hi * d_hi)
            acc = bias_inv[pl.ds(go, LANES)] + bias_tic[pl.ds(go, LANES)]
            acc = acc + bias_dat[pl.ds(go, LANES)] + gbv
            rbase = lanes * LANES
            for col in range(LANES):
                acc = acc + plsc.load_gather(part_v, [rbase + col])
            out_v[pl.ds(go, LANES)] = acc
            return c

        lax.fori_loop(0, HGROUPS, group, 0)
        return carry

    b1.wait()
    b2.wait()
    b3.wait()
    lax.fori_loop(0, 2, do_half, 0)
    pltpu.sync_copy(out_v, out.at[pl.ds(base, BPW)])


def kernel(investor, ticker, date, investor_factors, ticker_factors,
           date_factors, investor_bias, ticker_bias, date_bias, global_bias):
    gb16 = jnp.broadcast_to(global_bias.astype(jnp.float32), (LANES,))
    inv = investor.astype(jnp.int32)
    tic = ticker.astype(jnp.int32)
    dat = date.astype(jnp.int32)

    def submap(i, bk_log2):
        s = (i >> bk_log2) & 3
        row = ((i >> (bk_log2 + 2)) << bk_log2) | (i & ((1 << bk_log2) - 1))
        return s, row

    s_i, ki = submap(inv, 11)
    s_t, kt = submap(tic, 11)
    s_d, kd = submap(dat, 8)

    r_inv = _make_repack(1000000, 2048)(investor_factors.T)
    r_tic = _make_repack(100000, 2048)(ticker_factors.T)
    r_dat = _make_repack(1000, 256)(date_factors.T)

    return _mf_sc(
        r_inv, r_tic, r_dat,
        investor_bias.reshape(-1),
        ticker_bias.reshape(-1),
        date_bias.reshape(-1),
        gb16,
        ki, kt, kd, s_i, s_t, s_d, inv, tic, dat,
    )


# speedup vs baseline: 1.5831x; 1.0054x over previous
import functools

import jax
import jax.numpy as jnp
from jax import lax
from jax.experimental import pallas as pl
from jax.experimental.pallas import tpu as pltpu
from jax.experimental.pallas import tpu_sc as plsc

NUM_FACTORS = 32
BATCH = 16384
NUM_CORES = 2
NUM_SUBCORES = 16
NUM_WORKERS = NUM_CORES * NUM_SUBCORES
BPW = BATCH // NUM_WORKERS
LANES = 16
HALF = BPW // 2
HGROUPS = HALF // LANES

_mesh = plsc.VectorSubcoreMesh(core_axis_name="c", subcore_axis_name="s")


def _make_repack(n_rows: int, bk: int):
    grid = -(-n_rows // (4 * bk))

    def body(x, out):
        xt = jnp.swapaxes(x[...], 0, 1)
        out[...] = jnp.concatenate(
            [xt[s * bk:(s + 1) * bk, :] for s in range(4)], axis=1)

    return pl.pallas_call(
        body,
        grid=(grid,),
        in_specs=[pl.BlockSpec((NUM_FACTORS, 4 * bk), lambda g: (0, g))],
        out_specs=pl.BlockSpec((bk, 128), lambda g: (g, 0)),
        out_shape=jax.ShapeDtypeStruct((grid * bk, 128), jnp.float32),
    )


@functools.partial(
    pl.kernel,
    out_type=jax.ShapeDtypeStruct((BATCH,), jnp.float32),
    mesh=_mesh,
    scratch_types=[
        pltpu.VMEM((BPW,), jnp.int32),
        pltpu.VMEM((BPW,), jnp.int32),
        pltpu.VMEM((BPW,), jnp.int32),
        pltpu.VMEM((BPW,), jnp.int32),
        pltpu.VMEM((BPW,), jnp.int32),
        pltpu.VMEM((BPW,), jnp.int32),
        pltpu.VMEM((BPW,), jnp.int32),
        pltpu.VMEM((BPW,), jnp.int32),
        pltpu.VMEM((BPW,), jnp.int32),
        pltpu.VMEM((HALF, 128), jnp.float32),
        pltpu.VMEM((HALF, 128), jnp.float32),
        pltpu.VMEM((HALF, 128), jnp.float32),
        pltpu.VMEM((BPW,), jnp.float32),
        pltpu.VMEM((BPW,), jnp.float32),
        pltpu.VMEM((BPW,), jnp.float32),
        pltpu.VMEM((LANES,), jnp.float32),
        pltpu.VMEM((BPW,), jnp.float32),
        pltpu.VMEM((LANES * LANES,), jnp.float32),
        pltpu.SemaphoreType.DMA,
        pltpu.SemaphoreType.DMA,
    ],
    compiler_params=pltpu.CompilerParams(
        needs_layout_passes=False, use_tc_tiling_on_sc=False
    ),
)
def _mf_sc(r_inv, r_tic, r_dat, inv_b, tic_b, dat_b, gb, ki, kt, kd,
           si, st, sd, oi_h, ot_h, od_h, out, idx_inv, idx_tic, idx_dat,
           sub_inv, sub_tic, sub_dat, oidx_inv, oidx_tic, oidx_dat,
           rows_inv, rows_tic, rows_dat, bias_inv, bias_tic,
           bias_dat, gb_v, out_v, part_v, semb, semr):
    wid = lax.axis_index("s") * NUM_CORES + lax.axis_index("c")
    base = wid * BPW

    pltpu.sync_copy(ki.at[pl.ds(base, BPW)], idx_inv)
    pltpu.sync_copy(kt.at[pl.ds(base, BPW)], idx_tic)
    pltpu.sync_copy(kd.at[pl.ds(base, BPW)], idx_dat)
    pltpu.sync_copy(si.at[pl.ds(base, BPW)], sub_inv)
    pltpu.sync_copy(st.at[pl.ds(base, BPW)], sub_tic)
    pltpu.sync_copy(sd.at[pl.ds(base, BPW)], sub_dat)
    pltpu.sync_copy(oi_h.at[pl.ds(base, BPW)], oidx_inv)
    pltpu.sync_copy(ot_h.at[pl.ds(base, BPW)], oidx_tic)
    pltpu.sync_copy(od_h.at[pl.ds(base, BPW)], oidx_dat)
    pltpu.sync_copy(gb, gb_v)

    b1 = pltpu.async_copy(inv_b.at[oidx_inv], bias_inv, semb)
    b2 = pltpu.async_copy(tic_b.at[oidx_tic], bias_tic, semb)
    b3 = pltpu.async_copy(dat_b.at[oidx_dat], bias_dat, semb)

    gbv = gb_v[...]

    def do_half(h, carry):
        hbase = h * HALF
        g1 = pltpu.async_copy(
            r_inv.at[idx_inv.at[pl.ds(hbase, HALF)]], rows_inv, semr)
        g2 = pltpu.async_copy(
            r_tic.at[idx_tic.at[pl.ds(hbase, HALF)]], rows_tic, semr)
        g3 = pltpu.async_copy(
            r_dat.at[idx_dat.at[pl.ds(hbase, HALF)]], rows_dat, semr)
        g1.wait()
        g2.wait()
        g3.wait()

        lanes = lax.iota(jnp.int32, LANES)

        def group(g, c):
            o = g * LANES
            go = hbase + o
            sv_i = sub_inv[pl.ds(go, LANES)]
            sv_t = sub_tic[pl.ds(go, LANES)]
            sv_d = sub_dat[pl.ds(go, LANES)]
            for j in range(LANES):
                e = o + j
                jv = jnp.full((LANES,), j, jnp.int32)
                spl_i = jnp.take_along_axis(sv_i, jv, axis=0,
                                            mode="promise_in_bounds")
                spl_t = jnp.take_along_axis(sv_t, jv, axis=0,
                                            mode="promise_in_bounds")
                spl_d = jnp.take_along_axis(sv_d, jv, axis=0,
                                            mode="promise_in_bounds")

                def pick(rows, spl, off):
                    v = rows[e, pl.ds(off, LANES)]
                    for s in range(1, 4):
                        v = jnp.where(spl == s,
                                      rows[e, pl.ds(32 * s + off, LANES)], v)
                    return v

                a_lo = pick(rows_inv, spl_i, 0)
                a_hi = pick(rows_inv, spl_i, LANES)
                t_lo = pick(rows_tic, spl_t, 0)
                t_hi = pick(rows_tic, spl_t, LANES)
                d_lo = pick(rows_dat, spl_d, 0)
                d_hi = pick(rows_dat, spl_d, LANES)
                part_v[pl.ds(j * LANES, LANES)] = (
                    a_lo * t_lo * d_lo + a_hi * t_hi * d_hi)
            acc = bias_inv[pl.ds(go, LANES)] + bias_tic[pl.ds(go, LANES)]
            acc = acc + bias_dat[pl.ds(go, LANES)] + gbv
            rbase = lanes * LANES
            for col in range(LANES):
                acc = acc + plsc.load_gather(part_v, [rbase + col])
            out_v[pl.ds(go, LANES)] = acc
            return c

        lax.fori_loop(0, HGROUPS, group, 0)
        return carry

    b1.wait()
    b2.wait()
    b3.wait()
    lax.fori_loop(0, 2, do_half, 0)
    pltpu.sync_copy(out_v, out.at[pl.ds(base, BPW)])


def kernel(investor, ticker, date, investor_factors, ticker_factors,
           date_factors, investor_bias, ticker_bias, date_bias, global_bias):
    gb16 = jnp.broadcast_to(global_bias.astype(jnp.float32), (LANES,))
    inv = investor.astype(jnp.int32)
    tic = ticker.astype(jnp.int32)
    dat = date.astype(jnp.int32)

    def submap(i, bk_log2):
        s = (i >> bk_log2) & 3
        row = ((i >> (bk_log2 + 2)) << bk_log2) | (i & ((1 << bk_log2) - 1))
        return s, row

    s_i, ki = submap(inv, 12)
    s_t, kt = submap(tic, 12)
    s_d, kd = submap(dat, 8)

    r_inv = _make_repack(1000000, 4096)(investor_factors.T)
    r_tic = _make_repack(100000, 4096)(ticker_factors.T)
    r_dat = _make_repack(1000, 256)(date_factors.T)

    return _mf_sc(
        r_inv, r_tic, r_dat,
        investor_bias.reshape(-1),
        ticker_bias.reshape(-1),
        date_bias.reshape(-1),
        gb16,
        ki, kt, kd, s_i, s_t, s_d, inv, tic, dat,
    )
